# baseline (speedup 1.0000x reference)
import functools

import jax
import jax.numpy as jnp
from jax import lax
from jax.experimental import pallas as pl
from jax.experimental.pallas import tpu as pltpu
from jax.experimental.pallas import tpu_sc as plsc

_N_DIMS = 64
_SEQ_LEN = 200
_BATCH = 4096
_NW = 32
_BC = 128
_NBC = _BATCH // _BC
_NTASK = _SEQ_LEN * _NBC
_TPW = _NTASK // _NW

_mesh = plsc.VectorSubcoreMesh(core_axis_name="c", subcore_axis_name="s")


@functools.partial(
    pl.kernel,
    mesh=_mesh,
    out_type=jax.ShapeDtypeStruct((_SEQ_LEN, _N_DIMS, _BATCH), jnp.float32),
    scratch_types=[
        pltpu.VMEM((2, _BC), jnp.int32),
        pltpu.VMEM((2, _BC), jnp.int32),
        pltpu.VMEM((2, _BC, _N_DIMS), jnp.float32),
        pltpu.VMEM((2, _BC, _N_DIMS), jnp.float32),
        pltpu.VMEM((_SEQ_LEN // 2, 128), jnp.float32),
        pltpu.VMEM((2, _N_DIMS, _BC), jnp.float32),
        pltpu.SemaphoreType.DMA,
        pltpu.SemaphoreType.DMA,
        pltpu.SemaphoreType.DMA,
        pltpu.SemaphoreType.DMA,
        pltpu.SemaphoreType.DMA,
        pltpu.SemaphoreType.DMA,
        pltpu.SemaphoreType.DMA,
        pltpu.SemaphoreType.DMA,
        pltpu.SemaphoreType.DMA,
        pltpu.SemaphoreType.DMA,
    ],
    compiler_params=pltpu.CompilerParams(use_tc_tiling_on_sc=False,
                                         needs_layout_passes=False),
)
def _embed_kernel(eidx_hbm, cidx_hbm, etab_hbm, ctab_hbm, ptab_hbm, out_hbm,
                  eidx_v, cidx_v, erows_v, crows_v, pos_v, obuf_v,
                  sem_ie0, sem_ie1, sem_ic0, sem_ic1,
                  sem_ge0, sem_ge1, sem_gc0, sem_gc1, sem_o0, sem_o1):
    sem_ie = (sem_ie0, sem_ie1)
    sem_ic = (sem_ic0, sem_ic1)
    sem_ge = (sem_ge0, sem_ge1)
    sem_gc = (sem_gc0, sem_gc1)
    sem_o = (sem_o0, sem_o1)
    wid = lax.axis_index("s") * 2 + lax.axis_index("c")
    tbase = wid * _TPW
    pltpu.sync_copy(ptab_hbm, pos_v)

    def task_sb(t):
        gt = tbase + t
        return gt // _NBC, gt % _NBC

    def idx_fetch_start(t, b):
        s, bc = task_sb(t)
        pltpu.async_copy(eidx_hbm.at[s, pl.ds(bc * _BC, _BC)],
                         eidx_v.at[b], sem_ie[b])
        pltpu.async_copy(cidx_hbm.at[s, pl.ds(bc * _BC, _BC)],
                         cidx_v.at[b], sem_ic[b])

    def idx_wait(b):
        pltpu.make_async_copy(
            eidx_hbm.at[0, pl.ds(0, _BC)], eidx_v.at[b], sem_ie[b]).wait()
        pltpu.make_async_copy(
            cidx_hbm.at[0, pl.ds(0, _BC)], cidx_v.at[b], sem_ic[b]).wait()

    def gather_start(b):
        pltpu.async_copy(etab_hbm.at[eidx_v.at[b]], erows_v.at[b], sem_ge[b])
        pltpu.async_copy(ctab_hbm.at[cidx_v.at[b]], crows_v.at[b], sem_gc[b])

    def gather_wait(b):
        pltpu.make_async_copy(
            etab_hbm.at[eidx_v.at[0]], erows_v.at[b], sem_ge[b]).wait()
        pltpu.make_async_copy(
            ctab_hbm.at[cidx_v.at[0]], crows_v.at[b], sem_gc[b]).wait()

    def out_start(t, b):
        s, bc = task_sb(t)
        pltpu.async_copy(obuf_v.at[b],
                         out_hbm.at[s, slice(None), pl.ds(bc * _BC, _BC)],
                         sem_o[b])

    def out_wait(b):
        pltpu.make_async_copy(
            obuf_v.at[b], out_hbm.at[0, slice(None), pl.ds(0, _BC)],
            sem_o[b]).wait()

    idx_fetch_start(0, 0)
    idx_fetch_start(1, 1)
    idx_wait(0)
    gather_start(0)

    _iota = jnp.arange(16, dtype=jnp.int32)
    _drows = tuple(_iota + g * 16 for g in range(_N_DIMS // 16))

    def body(half, _):
        for tb in range(2):
            t = half * 2 + tb
            nb = 1 - tb

            @pl.when(t < _TPW - 1)
            def _():
                idx_wait(nb)
                gather_start(nb)

            gather_wait(tb)

            @pl.when(t < _TPW - 2)
            def _():
                idx_fetch_start(t + 2, tb)

            @pl.when(t >= 2)
            def _():
                out_wait(tb)

            s, _bc = task_sb(t)
            s2 = s // 2
            pcol0 = (s % 2) * _N_DIMS
            prow = tuple(
                pos_v[s2, pl.ds(pcol0 + g * 16, 16)]
                for g in range(_N_DIMS // 16))
            erows = erows_v.at[tb]
            crows = crows_v.at[tb]
            obufT = obuf_v.at[tb]

            def r_body(r2, _):
                for u in range(4):
                    r = r2 * 4 + u
                    cvec = jnp.full((16,), r, jnp.int32)
                    for g in range(_N_DIMS // 16):
                        val = (erows[r, pl.ds(g * 16, 16)]
                               + crows[r, pl.ds(g * 16, 16)] + prow[g])
                        plsc.store_scatter(obufT, [_drows[g], cvec], val)
                return 0

            lax.fori_loop(0, _BC // 4, r_body, 0)
            out_start(t, tb)
        return 0

    lax.fori_loop(0, _TPW // 2, body, 0)
    out_wait(0)
    out_wait(1)


def kernel(exercises, categories, exercise_embed, category_embed, position_embed):
    eidx = exercises.astype(jnp.int32).T
    cidx = categories.astype(jnp.int32).T
    pos128 = position_embed.reshape(_SEQ_LEN // 2, 128)
    out = _embed_kernel(eidx, cidx, exercise_embed, category_embed, pos128)
    return out.transpose(2, 0, 1)


# speedup vs baseline: 1.5677x; 1.5677x over previous
import functools

import jax
import jax.numpy as jnp
from jax import lax
from jax.experimental import pallas as pl
from jax.experimental.pallas import tpu as pltpu
from jax.experimental.pallas import tpu_sc as plsc

_N_DIMS = 64
_SEQ_LEN = 200
_BATCH = 4096
_NW = 32
_BC = 128
_NBC = _BATCH // _BC
_NTASK = _SEQ_LEN * _NBC
_TPW = _NTASK // _NW

_mesh = plsc.VectorSubcoreMesh(core_axis_name="c", subcore_axis_name="s")


@functools.partial(
    pl.kernel,
    mesh=_mesh,
    out_type=jax.ShapeDtypeStruct((_SEQ_LEN, _N_DIMS, _BATCH), jnp.float32),
    scratch_types=[
        pltpu.VMEM((2, _BC), jnp.int32),
        pltpu.VMEM((2, _BC), jnp.int32),
        pltpu.VMEM((2, _BC, _N_DIMS), jnp.float32),
        pltpu.VMEM((2, _BC, _N_DIMS), jnp.float32),
        pltpu.VMEM((_SEQ_LEN // 2, 128), jnp.float32),
        pltpu.VMEM((2, _N_DIMS, _BC + 1), jnp.float32),
        pltpu.SemaphoreType.DMA,
        pltpu.SemaphoreType.DMA,
        pltpu.SemaphoreType.DMA,
        pltpu.SemaphoreType.DMA,
        pltpu.SemaphoreType.DMA,
        pltpu.SemaphoreType.DMA,
        pltpu.SemaphoreType.DMA,
        pltpu.SemaphoreType.DMA,
        pltpu.SemaphoreType.DMA,
        pltpu.SemaphoreType.DMA,
    ],
    compiler_params=pltpu.CompilerParams(use_tc_tiling_on_sc=False,
                                         needs_layout_passes=False),
)
def _embed_kernel(eidx_hbm, cidx_hbm, etab_hbm, ctab_hbm, ptab_hbm, out_hbm,
                  eidx_v, cidx_v, erows_v, crows_v, pos_v, obuf_v,
                  sem_ie0, sem_ie1, sem_ic0, sem_ic1,
                  sem_ge0, sem_ge1, sem_gc0, sem_gc1, sem_o0, sem_o1):
    sem_ie = (sem_ie0, sem_ie1)
    sem_ic = (sem_ic0, sem_ic1)
    sem_ge = (sem_ge0, sem_ge1)
    sem_gc = (sem_gc0, sem_gc1)
    sem_o = (sem_o0, sem_o1)
    wid = lax.axis_index("s") * 2 + lax.axis_index("c")
    tbase = wid * _TPW
    pltpu.sync_copy(ptab_hbm, pos_v)

    def task_sb(t):
        gt = tbase + t
        return gt // _NBC, gt % _NBC

    def idx_fetch_start(t, b):
        s, bc = task_sb(t)
        pltpu.async_copy(eidx_hbm.at[s, pl.ds(bc * _BC, _BC)],
                         eidx_v.at[b], sem_ie[b])
        pltpu.async_copy(cidx_hbm.at[s, pl.ds(bc * _BC, _BC)],
                         cidx_v.at[b], sem_ic[b])

    def idx_wait(b):
        pltpu.make_async_copy(
            eidx_hbm.at[0, pl.ds(0, _BC)], eidx_v.at[b], sem_ie[b]).wait()
        pltpu.make_async_copy(
            cidx_hbm.at[0, pl.ds(0, _BC)], cidx_v.at[b], sem_ic[b]).wait()

    def gather_start(b):
        pltpu.async_copy(etab_hbm.at[eidx_v.at[b]], erows_v.at[b], sem_ge[b])
        pltpu.async_copy(ctab_hbm.at[cidx_v.at[b]], crows_v.at[b], sem_gc[b])

    def gather_wait(b):
        pltpu.make_async_copy(
            etab_hbm.at[eidx_v.at[0]], erows_v.at[b], sem_ge[b]).wait()
        pltpu.make_async_copy(
            ctab_hbm.at[cidx_v.at[0]], crows_v.at[b], sem_gc[b]).wait()

    def out_start(t, b):
        s, bc = task_sb(t)
        pltpu.async_copy(obuf_v.at[b, slice(None), pl.ds(0, _BC)],
                         out_hbm.at[s, slice(None), pl.ds(bc * _BC, _BC)],
                         sem_o[b])

    def out_wait(b):
        pltpu.make_async_copy(
            obuf_v.at[b, slice(None), pl.ds(0, _BC)],
            out_hbm.at[0, slice(None), pl.ds(0, _BC)],
            sem_o[b]).wait()

    idx_fetch_start(0, 0)
    idx_fetch_start(1, 1)
    idx_wait(0)
    gather_start(0)

    _iota = jnp.arange(16, dtype=jnp.int32)
    _drows = tuple(_iota + g * 16 for g in range(_N_DIMS // 16))

    def body(half, _):
        for tb in range(2):
            t = half * 2 + tb
            nb = 1 - tb

            @pl.when(t < _TPW - 1)
            def _():
                idx_wait(nb)
                gather_start(nb)

            gather_wait(tb)

            @pl.when(t < _TPW - 2)
            def _():
                idx_fetch_start(t + 2, tb)

            @pl.when(t >= 2)
            def _():
                out_wait(tb)

            s, _bc = task_sb(t)
            s2 = s // 2
            pcol0 = (s % 2) * _N_DIMS
            prow = tuple(
                pos_v[s2, pl.ds(pcol0 + g * 16, 16)]
                for g in range(_N_DIMS // 16))
            erows = erows_v.at[tb]
            crows = crows_v.at[tb]
            obufT = obuf_v.at[tb]

            def r_body(r2, _):
                for u in range(4):
                    r = r2 * 4 + u
                    cvec = jnp.full((16,), r, jnp.int32)
                    for g in range(_N_DIMS // 16):
                        val = (erows[r, pl.ds(g * 16, 16)]
                               + crows[r, pl.ds(g * 16, 16)] + prow[g])
                        plsc.store_scatter(obufT, [_drows[g], cvec], val)
                return 0

            lax.fori_loop(0, _BC // 4, r_body, 0)
            out_start(t, tb)
        return 0

    lax.fori_loop(0, _TPW // 2, body, 0)
    out_wait(0)
    out_wait(1)


def kernel(exercises, categories, exercise_embed, category_embed, position_embed):
    eidx = exercises.astype(jnp.int32).T
    cidx = categories.astype(jnp.int32).T
    pos128 = position_embed.reshape(_SEQ_LEN // 2, 128)
    out = _embed_kernel(eidx, cidx, exercise_embed, category_embed, pos128)
    return out.transpose(2, 0, 1)


# speedup vs baseline: 2.5771x; 1.6439x over previous
import functools

import jax
import jax.numpy as jnp
from jax import lax
from jax.experimental import pallas as pl
from jax.experimental.pallas import tpu as pltpu
from jax.experimental.pallas import tpu_sc as plsc

_N_DIMS = 64
_SEQ_LEN = 200
_BATCH = 4096
_NW = 32
_BC = 128
_NBC = _BATCH // _BC
_NTASK = _SEQ_LEN * _NBC
_TPW = _NTASK // _NW

_mesh = plsc.VectorSubcoreMesh(core_axis_name="c", subcore_axis_name="s")


@functools.partial(
    pl.kernel,
    mesh=_mesh,
    out_type=jax.ShapeDtypeStruct((_SEQ_LEN, _N_DIMS, _BATCH), jnp.float32),
    scratch_types=[
        pltpu.VMEM((2, _BC), jnp.int32),
        pltpu.VMEM((2, _BC), jnp.int32),
        pltpu.VMEM((2, _BC, _N_DIMS), jnp.float32),
        pltpu.VMEM((2, _BC, _N_DIMS), jnp.float32),
        pltpu.VMEM((_SEQ_LEN // 2, 128), jnp.float32),
        pltpu.VMEM((2, _N_DIMS, _BC + 1), jnp.float32),
        pltpu.SemaphoreType.DMA,
        pltpu.SemaphoreType.DMA,
        pltpu.SemaphoreType.DMA,
        pltpu.SemaphoreType.DMA,
        pltpu.SemaphoreType.DMA,
        pltpu.SemaphoreType.DMA,
        pltpu.SemaphoreType.DMA,
        pltpu.SemaphoreType.DMA,
        pltpu.SemaphoreType.DMA,
        pltpu.SemaphoreType.DMA,
    ],
    compiler_params=pltpu.CompilerParams(use_tc_tiling_on_sc=False,
                                         needs_layout_passes=False),
)
def _embed_kernel(eidx_hbm, cidx_hbm, etab_hbm, ctab_hbm, ptab_hbm, out_hbm,
                  eidx_v, cidx_v, erows_v, crows_v, pos_v, obuf_v,
                  sem_ie0, sem_ie1, sem_ic0, sem_ic1,
                  sem_ge0, sem_ge1, sem_gc0, sem_gc1, sem_o0, sem_o1):
    sem_ie = (sem_ie0, sem_ie1)
    sem_ic = (sem_ic0, sem_ic1)
    sem_ge = (sem_ge0, sem_ge1)
    sem_gc = (sem_gc0, sem_gc1)
    sem_o = (sem_o0, sem_o1)
    wid = lax.axis_index("s") * 2 + lax.axis_index("c")
    tbase = wid * _TPW
    pltpu.sync_copy(ptab_hbm, pos_v)

    def task_sb(t):
        gt = tbase + t
        return gt // _NBC, gt % _NBC

    def idx_fetch_start(t, b):
        s, bc = task_sb(t)
        pltpu.async_copy(eidx_hbm.at[s, pl.ds(bc * _BC, _BC)],
                         eidx_v.at[b], sem_ie[b])
        pltpu.async_copy(cidx_hbm.at[s, pl.ds(bc * _BC, _BC)],
                         cidx_v.at[b], sem_ic[b])

    def idx_wait(b):
        pltpu.make_async_copy(
            eidx_hbm.at[0, pl.ds(0, _BC)], eidx_v.at[b], sem_ie[b]).wait()
        pltpu.make_async_copy(
            cidx_hbm.at[0, pl.ds(0, _BC)], cidx_v.at[b], sem_ic[b]).wait()

    def gather_start(b):
        pltpu.async_copy(etab_hbm.at[eidx_v.at[b]], erows_v.at[b], sem_ge[b])
        pltpu.async_copy(ctab_hbm.at[cidx_v.at[b]], crows_v.at[b], sem_gc[b])

    def gather_wait(b):
        pltpu.make_async_copy(
            etab_hbm.at[eidx_v.at[0]], erows_v.at[b], sem_ge[b]).wait()
        pltpu.make_async_copy(
            ctab_hbm.at[cidx_v.at[0]], crows_v.at[b], sem_gc[b]).wait()

    def out_start(t, b):
        s, bc = task_sb(t)
        pltpu.async_copy(obuf_v.at[b, slice(None), pl.ds(0, _BC)],
                         out_hbm.at[s, slice(None), pl.ds(bc * _BC, _BC)],
                         sem_o[b])

    def out_wait(b):
        pltpu.make_async_copy(
            obuf_v.at[b, slice(None), pl.ds(0, _BC)],
            out_hbm.at[0, slice(None), pl.ds(0, _BC)],
            sem_o[b]).wait()

    idx_fetch_start(0, 0)
    idx_fetch_start(1, 1)
    idx_wait(0)
    gather_start(0)

    _iota = jnp.arange(16, dtype=jnp.int32)
    _drows = tuple(_iota + g * 16 for g in range(_N_DIMS // 16))

    def body(half, _):
        for tb in range(2):
            t = half * 2 + tb
            nb = 1 - tb

            @pl.when(t < _TPW - 1)
            def _():
                idx_wait(nb)
                gather_start(nb)

            gather_wait(tb)

            @pl.when(t < _TPW - 2)
            def _():
                idx_fetch_start(t + 2, tb)

            @pl.when(t >= 2)
            def _():
                out_wait(tb)

            s, _bc = task_sb(t)
            s2 = s // 2
            pcol0 = (s % 2) * _N_DIMS
            prow = tuple(
                pos_v[s2, pl.ds(pcol0 + g * 16, 16)]
                for g in range(_N_DIMS // 16))
            erows = erows_v.at[tb]
            crows = crows_v.at[tb]
            obufT = obuf_v.at[tb]

            @plsc.parallel_loop(0, _BC, unroll=4)
            def r_body(r):
                cvec = jnp.full((16,), r, jnp.int32)
                for g in range(_N_DIMS // 16):
                    val = (erows[r, pl.ds(g * 16, 16)]
                           + crows[r, pl.ds(g * 16, 16)] + prow[g])
                    plsc.store_scatter(obufT, [_drows[g], cvec], val)
            out_start(t, tb)
        return 0

    lax.fori_loop(0, _TPW // 2, body, 0)
    out_wait(0)
    out_wait(1)


def kernel(exercises, categories, exercise_embed, category_embed, position_embed):
    eidx = exercises.astype(jnp.int32).T
    cidx = categories.astype(jnp.int32).T
    pos128 = position_embed.reshape(_SEQ_LEN // 2, 128)
    out = _embed_kernel(eidx, cidx, exercise_embed, category_embed, pos128)
    return out.transpose(2, 0, 1)


# speedup vs baseline: 2.5839x; 1.0027x over previous
import functools

import jax
import jax.numpy as jnp
from jax import lax
from jax.experimental import pallas as pl
from jax.experimental.pallas import tpu as pltpu
from jax.experimental.pallas import tpu_sc as plsc

_N_DIMS = 64
_SEQ_LEN = 200
_BATCH = 4096
_NW = 32
_BC = 128
_NBC = _BATCH // _BC
_NTASK = _SEQ_LEN * _NBC
_TPW = _NTASK // _NW

_mesh = plsc.VectorSubcoreMesh(core_axis_name="c", subcore_axis_name="s")


@functools.partial(
    pl.kernel,
    mesh=_mesh,
    out_type=jax.ShapeDtypeStruct((_SEQ_LEN, _N_DIMS, _BATCH), jnp.float32),
    scratch_types=[
        pltpu.VMEM((2, _BC), jnp.int32),
        pltpu.VMEM((2, _BC), jnp.int32),
        pltpu.VMEM((2, _BC, _N_DIMS), jnp.float32),
        pltpu.VMEM((2, _BC, _N_DIMS), jnp.float32),
        pltpu.VMEM((_SEQ_LEN // 2, 128), jnp.float32),
        pltpu.VMEM((2, _N_DIMS, _BC + 1), jnp.float32),
        pltpu.SemaphoreType.DMA,
        pltpu.SemaphoreType.DMA,
        pltpu.SemaphoreType.DMA,
        pltpu.SemaphoreType.DMA,
        pltpu.SemaphoreType.DMA,
        pltpu.SemaphoreType.DMA,
        pltpu.SemaphoreType.DMA,
        pltpu.SemaphoreType.DMA,
        pltpu.SemaphoreType.DMA,
        pltpu.SemaphoreType.DMA,
    ],
    compiler_params=pltpu.CompilerParams(use_tc_tiling_on_sc=False,
                                         needs_layout_passes=False,
                                         skip_device_barrier=True),
)
def _embed_kernel(eidx_hbm, cidx_hbm, etab_hbm, ctab_hbm, ptab_hbm, out_hbm,
                  eidx_v, cidx_v, erows_v, crows_v, pos_v, obuf_v,
                  sem_ie0, sem_ie1, sem_ic0, sem_ic1,
                  sem_ge0, sem_ge1, sem_gc0, sem_gc1, sem_o0, sem_o1):
    sem_ie = (sem_ie0, sem_ie1)
    sem_ic = (sem_ic0, sem_ic1)
    sem_ge = (sem_ge0, sem_ge1)
    sem_gc = (sem_gc0, sem_gc1)
    sem_o = (sem_o0, sem_o1)
    wid = lax.axis_index("s") * 2 + lax.axis_index("c")
    tbase = wid * _TPW
    pltpu.sync_copy(ptab_hbm, pos_v)

    def task_sb(t):
        gt = tbase + t
        return gt // _NBC, gt % _NBC

    def idx_fetch_start(t, b):
        s, bc = task_sb(t)
        pltpu.async_copy(eidx_hbm.at[s, pl.ds(bc * _BC, _BC)],
                         eidx_v.at[b], sem_ie[b])
        pltpu.async_copy(cidx_hbm.at[s, pl.ds(bc * _BC, _BC)],
                         cidx_v.at[b], sem_ic[b])

    def idx_wait(b):
        pltpu.make_async_copy(
            eidx_hbm.at[0, pl.ds(0, _BC)], eidx_v.at[b], sem_ie[b]).wait()
        pltpu.make_async_copy(
            cidx_hbm.at[0, pl.ds(0, _BC)], cidx_v.at[b], sem_ic[b]).wait()

    def gather_start(b):
        pltpu.async_copy(etab_hbm.at[eidx_v.at[b]], erows_v.at[b], sem_ge[b])
        pltpu.async_copy(ctab_hbm.at[cidx_v.at[b]], crows_v.at[b], sem_gc[b])

    def gather_wait(b):
        pltpu.make_async_copy(
            etab_hbm.at[eidx_v.at[0]], erows_v.at[b], sem_ge[b]).wait()
        pltpu.make_async_copy(
            ctab_hbm.at[cidx_v.at[0]], crows_v.at[b], sem_gc[b]).wait()

    def out_start(t, b):
        s, bc = task_sb(t)
        pltpu.async_copy(obuf_v.at[b, slice(None), pl.ds(0, _BC)],
                         out_hbm.at[s, slice(None), pl.ds(bc * _BC, _BC)],
                         sem_o[b])

    def out_wait(b):
        pltpu.make_async_copy(
            obuf_v.at[b, slice(None), pl.ds(0, _BC)],
            out_hbm.at[0, slice(None), pl.ds(0, _BC)],
            sem_o[b]).wait()

    idx_fetch_start(0, 0)
    idx_fetch_start(1, 1)
    idx_wait(0)
    gather_start(0)

    _iota = jnp.arange(16, dtype=jnp.int32)
    _drows = tuple(_iota + g * 16 for g in range(_N_DIMS // 16))

    def body(half, _):
        for tb in range(2):
            t = half * 2 + tb
            nb = 1 - tb

            @pl.when(t < _TPW - 1)
            def _():
                idx_wait(nb)
                gather_start(nb)

            gather_wait(tb)

            @pl.when(t < _TPW - 2)
            def _():
                idx_fetch_start(t + 2, tb)

            @pl.when(t >= 2)
            def _():
                out_wait(tb)

            s, _bc = task_sb(t)
            s2 = s // 2
            pcol0 = (s % 2) * _N_DIMS
            prow = tuple(
                pos_v[s2, pl.ds(pcol0 + g * 16, 16)]
                for g in range(_N_DIMS // 16))
            erows = erows_v.at[tb]
            crows = crows_v.at[tb]
            obufT = obuf_v.at[tb]

            @plsc.parallel_loop(0, _BC, unroll=4)
            def r_body(r):
                cvec = jnp.full((16,), r, jnp.int32)
                for g in range(_N_DIMS // 16):
                    val = (erows[r, pl.ds(g * 16, 16)]
                           + crows[r, pl.ds(g * 16, 16)] + prow[g])
                    plsc.store_scatter(obufT, [_drows[g], cvec], val)
            out_start(t, tb)
        return 0

    lax.fori_loop(0, _TPW // 2, body, 0)
    out_wait(0)
    out_wait(1)


def kernel(exercises, categories, exercise_embed, category_embed, position_embed):
    eidx = exercises.astype(jnp.int32).T
    cidx = categories.astype(jnp.int32).T
    pos128 = position_embed.reshape(_SEQ_LEN // 2, 128)
    out = _embed_kernel(eidx, cidx, exercise_embed, category_embed, pos128)
    return out.transpose(2, 0, 1)
